# baseline (speedup 1.0000x reference)
import functools

import jax
import jax.numpy as jnp
from jax import lax
from jax.experimental import pallas as pl
from jax.experimental.pallas import tpu as pltpu
from jax.experimental.pallas import tpu_sc as plsc

VOCAB = 16
D = 16
CHUNK = 1024
GROUPS = CHUNK // 16


def _h_body(e_ref, w_ref, b_ref, h_ref):
    h = lax.dot_general(
        e_ref[...], w_ref[...],
        (((1,), (1,)), ((), ())),
        preferred_element_type=jnp.float32,
    )
    h_ref[...] = h + b_ref[...]


def _compute_h(embed_table, W, b):
    b_mat = jnp.broadcast_to(b.reshape(1, D), (VOCAB, D))
    return pl.pallas_call(
        _h_body,
        out_shape=jax.ShapeDtypeStruct((VOCAB, D), jnp.float32),
    )(embed_table, W, b_mat)


def _sc_gather(ids, h_flat):
    n = ids.shape[0]
    info = plsc.get_sparse_core_info()
    nc, ns = info.num_cores, info.num_subcores
    nw = nc * ns
    per_w = n // nw
    assert per_w * nw == n and per_w % CHUNK == 0
    n_chunks = per_w // CHUNK
    assert n_chunks % 2 == 0
    n_iter = n_chunks // 2

    mesh = plsc.VectorSubcoreMesh(core_axis_name="c", subcore_axis_name="s")

    @functools.partial(
        pl.kernel,
        out_type=[
            jax.ShapeDtypeStruct((n * D,), jnp.float32),
            jax.ShapeDtypeStruct((n * D,), jnp.float32),
        ],
        mesh=mesh,
        scratch_types=[
            pltpu.VMEM((VOCAB * D,), jnp.float32),
            pltpu.VMEM((CHUNK,), jnp.int32),
            pltpu.VMEM((CHUNK,), jnp.int32),
            pltpu.VMEM((CHUNK * D,), jnp.float32),
            pltpu.VMEM((CHUNK * D,), jnp.float32),
            pltpu.VMEM((CHUNK * D,), jnp.float32),
            pltpu.VMEM((CHUNK * D,), jnp.float32),
            pltpu.SemaphoreType.DMA,
            pltpu.SemaphoreType.DMA,
            pltpu.SemaphoreType.DMA,
            pltpu.SemaphoreType.DMA,
        ],
        compiler_params=pltpu.CompilerParams(
            needs_layout_passes=False, use_tc_tiling_on_sc=True),
    )
    def k(ids_hbm, h_hbm, hid_hbm, log_hbm,
          h_v, iv0, iv1, hv0, hv1, lv0, lv1, si0, si1, sw0, sw1):
        wid = lax.axis_index("s") * nc + lax.axis_index("c")
        base = wid * per_w
        idbufs = (iv0, iv1)
        hbufs = (hv0, hv1)
        lbufs = (lv0, lv1)
        isems = (si0, si1)
        wsems = (sw0, sw1)

        pltpu.sync_copy(h_hbm, h_v)
        lane16 = lax.iota(jnp.int32, 16) * D

        def ids_src(c):
            return ids_hbm.at[pl.ds(base + c * CHUNK, CHUNK)]

        def out_dst(hbm, c):
            return hbm.at[pl.ds((base + c * CHUNK) * D, CHUNK * D)]

        pltpu.async_copy(ids_src(0), iv0, si0)

        def iter_body(i, carry):
            for b in range(2):
                c = i * 2 + b
                pltpu.make_async_copy(ids_src(c), idbufs[b], isems[b]).wait()

                if b == 0:
                    pltpu.async_copy(ids_src(c + 1), idbufs[1], isems[1])
                else:
                    @pl.when(i < n_iter - 1)
                    def _():
                        pltpu.async_copy(ids_src(c + 1), idbufs[0], isems[0])

                @pl.when(i >= 1)
                def _():
                    pltpu.make_async_copy(
                        hbufs[b], out_dst(hid_hbm, c), wsems[b]).wait()
                    pltpu.make_async_copy(
                        lbufs[b], out_dst(log_hbm, c), wsems[b]).wait()

                hid_v, log_v, ids_v = hbufs[b], lbufs[b], idbufs[b]

                def group_body(g, carry2):
                    idv = ids_v[pl.ds(g * 16, 16)]
                    bi = idv * D
                    g0 = plsc.load_gather(h_v, (bi,))
                    pos0 = lane16 + g * (16 * D)
                    for j in range(D):
                        r = plsc.load_gather(h_v, (bi + j,))
                        plsc.store_scatter(hid_v, (pos0 + j,), r)
                        plsc.store_scatter(log_v, (pos0 + j,), g0)
                    return carry2

                lax.fori_loop(0, GROUPS, group_body, 0, unroll=4)

                pltpu.async_copy(hid_v, out_dst(hid_hbm, c), wsems[b])
                pltpu.async_copy(log_v, out_dst(log_hbm, c), wsems[b])
            return carry

        lax.fori_loop(0, n_iter, iter_body, 0, unroll=False)

        for b in range(2):
            c = n_chunks - 2 + b
            pltpu.make_async_copy(
                hbufs[b], out_dst(hid_hbm, c), wsems[b]).wait()
            pltpu.make_async_copy(
                lbufs[b], out_dst(log_hbm, c), wsems[b]).wait()

    return k(ids, h_flat)


def kernel(input_ids, embed_table, W, b):
    bsz, seq = input_ids.shape
    ids = input_ids.reshape(-1).astype(jnp.int32)
    h = _compute_h(embed_table, W, b)
    hid_flat, log_flat = _sc_gather(ids, h.reshape(-1))
    hidden = hid_flat.reshape(bsz, seq, D)
    logits = log_flat.reshape(bsz, seq, D)
    return (logits, hidden)


# speedup vs baseline: 4.6079x; 4.6079x over previous
import functools

import jax
import jax.numpy as jnp
from jax import lax
from jax.experimental import pallas as pl
from jax.experimental.pallas import tpu as pltpu
from jax.experimental.pallas import tpu_sc as plsc

VOCAB = 16
D = 16
B = 16384
L = 200
LT = L // 8
NBT = B // 128


def _h_body(e_ref, w_ref, b_ref, h_ref):
    h = lax.dot_general(
        e_ref[...], w_ref[...],
        (((1,), (1,)), ((), ())),
        preferred_element_type=jnp.float32,
    )
    h_ref[...] = h + b_ref[...]


def _compute_h(embed_table, W, b):
    b_mat = jnp.broadcast_to(b.reshape(1, D), (VOCAB, D))
    return pl.pallas_call(
        _h_body,
        out_shape=jax.ShapeDtypeStruct((VOCAB, D), jnp.float32),
    )(embed_table, W, b_mat)


def _sc_gather(ids_phys, h_flat):
    n = ids_phys.shape[0]
    assert n == B * L
    info = plsc.get_sparse_core_info()
    nc, ns = info.num_cores, info.num_subcores
    nw = nc * ns
    btw = NBT // nw
    assert btw * nw == NBT
    blk = btw * 8 * 128

    mesh = plsc.VectorSubcoreMesh(core_axis_name="c", subcore_axis_name="s")

    @functools.partial(
        pl.kernel,
        out_type=[
            jax.ShapeDtypeStruct((n * D,), jnp.float32),
            jax.ShapeDtypeStruct((n * D,), jnp.float32),
        ],
        mesh=mesh,
        scratch_types=[
            pltpu.VMEM((VOCAB * D,), jnp.float32),
            pltpu.VMEM((blk,), jnp.int32),
            pltpu.VMEM((2 * btw * 1024,), jnp.float32),
            pltpu.VMEM((2 * btw * 1024,), jnp.float32),
            pltpu.VMEM((2 * btw * 1024,), jnp.float32),
            pltpu.VMEM((2 * btw * 1024,), jnp.float32),
            pltpu.SemaphoreType.DMA,
            pltpu.SemaphoreType.DMA,
        ],
        compiler_params=pltpu.CompilerParams(
            needs_layout_passes=False, use_tc_tiling_on_sc=True),
    )
    def k(ids_hbm, h_hbm, hid_hbm, log_hbm,
          h_v, ids_v, hb0, hb1, lb0, lb1, sw0, sw1):
        wid = lax.axis_index("s") * nc + lax.axis_index("c")
        bt0 = wid * btw
        hbufs = (hb0, hb1)
        lbufs = (lb0, lb1)
        wsems = (sw0, sw1)
        half = btw * 1024

        pltpu.sync_copy(h_hbm, h_v)

        def out_dst(hbm, l, jt):
            return hbm.at[pl.ds(((l * 2 + jt) * NBT + bt0) * 1024, half)]

        def drain(p, l):
            for jt in range(2):
                pltpu.make_async_copy(
                    hbufs[p].at[pl.ds(jt * half, half)],
                    out_dst(hid_hbm, l, jt), wsems[p]).wait()
                pltpu.make_async_copy(
                    lbufs[p].at[pl.ds(jt * half, half)],
                    out_dst(log_hbm, l, jt), wsems[p]).wait()

        def lt_body(lt, carry):
            pltpu.sync_copy(
                ids_hbm.at[pl.ds((lt * NBT + bt0) * 1024, blk)], ids_v)
            for ll in range(8):
                p = ll % 2
                l = lt * 8 + ll
                hid_l, log_l = hbufs[p], lbufs[p]

                if ll >= 2:
                    drain(p, l)
                else:
                    @pl.when(lt >= 1)
                    def _():
                        drain(p, l)

                def t_body(t, c2):
                    base_t = (t // 8) * 1024 + (t % 8) * 16
                    idv = ids_v[pl.ds(base_t + ll * 128, 16)]
                    bi = idv * D
                    g0 = plsc.load_gather(h_v, (bi,))
                    for j in range(D):
                        off = (j // 8) * half + base_t + (j % 8) * 128
                        if j == 0:
                            r = g0
                        else:
                            r = plsc.load_gather(h_v, (bi + j,))
                        hid_l[pl.ds(off, 16)] = r
                        log_l[pl.ds(off, 16)] = g0
                    return c2

                lax.fori_loop(0, btw * 8, t_body, 0, unroll=2)

                for jt in range(2):
                    pltpu.async_copy(
                        hid_l.at[pl.ds(jt * half, half)],
                        out_dst(hid_hbm, l, jt), wsems[p])
                    pltpu.async_copy(
                        log_l.at[pl.ds(jt * half, half)],
                        out_dst(log_hbm, l, jt), wsems[p])
            return carry

        lax.fori_loop(0, LT, lt_body, 0, unroll=False)

        for ll in (6, 7):
            drain(ll % 2, (LT - 1) * 8 + ll)

    return k(ids_phys, h_flat)


def kernel(input_ids, embed_table, W, b):
    ids_phys = (input_ids.T.reshape(LT, 8, NBT, 128)
                .transpose(0, 2, 1, 3).reshape(-1).astype(jnp.int32))
    h = _compute_h(embed_table, W, b)
    hid_flat, log_flat = _sc_gather(ids_phys, h.reshape(-1))

    def unphys(flat):
        return (flat.reshape(L, 2, NBT, 8, 128)
                .transpose(2, 4, 0, 1, 3).reshape(B, L, D))

    return (unphys(log_flat), unphys(hid_flat))


# speedup vs baseline: 10.7513x; 2.3332x over previous
import functools

import jax
import jax.numpy as jnp
from jax import lax
from jax.experimental import pallas as pl
from jax.experimental.pallas import tpu as pltpu
from jax.experimental.pallas import tpu_sc as plsc

VOCAB = 16
D = 16
B = 16384
L = 200
LT = L // 8
NBT = B // 128


def _h_body(e_ref, w_ref, b_ref, h_ref):
    h = lax.dot_general(
        e_ref[...], w_ref[...],
        (((1,), (1,)), ((), ())),
        preferred_element_type=jnp.float32,
    )
    h_ref[...] = h + b_ref[...]


def _compute_h(embed_table, W, b):
    b_mat = jnp.broadcast_to(b.reshape(1, D), (VOCAB, D))
    return pl.pallas_call(
        _h_body,
        out_shape=jax.ShapeDtypeStruct((VOCAB, D), jnp.float32),
    )(embed_table, W, b_mat)


def _sc_gather(ids_phys, h_flat):
    n = ids_phys.shape[0]
    assert n == B * L
    info = plsc.get_sparse_core_info()
    nc, ns = info.num_cores, info.num_subcores
    nw = nc * ns
    btw = NBT // nw
    assert btw * nw == NBT
    blk = btw * 8 * 128

    mesh = plsc.VectorSubcoreMesh(core_axis_name="c", subcore_axis_name="s")

    @functools.partial(
        pl.kernel,
        out_type=[
            jax.ShapeDtypeStruct((n * D,), jnp.float32),
            jax.ShapeDtypeStruct((n * D,), jnp.float32),
        ],
        mesh=mesh,
        scratch_types=[
            pltpu.VMEM((VOCAB * D,), jnp.float32),
            pltpu.VMEM((blk,), jnp.int32),
            pltpu.VMEM((2 * btw * 1024,), jnp.float32),
            pltpu.VMEM((2 * btw * 1024,), jnp.float32),
            pltpu.VMEM((2 * btw * 1024,), jnp.float32),
            pltpu.VMEM((2 * btw * 1024,), jnp.float32),
            pltpu.SemaphoreType.DMA,
            pltpu.SemaphoreType.DMA,
        ],
        compiler_params=pltpu.CompilerParams(
            needs_layout_passes=False, use_tc_tiling_on_sc=True),
    )
    def k(ids_hbm, h_hbm, hid_hbm, log_hbm,
          h_v, ids_v, hb0, hb1, lb0, lb1, sw0, sw1):
        wid = lax.axis_index("s") * nc + lax.axis_index("c")
        bt0 = wid * btw
        hbufs = (hb0, hb1)
        lbufs = (lb0, lb1)
        wsems = (sw0, sw1)
        half = btw * 1024

        pltpu.sync_copy(h_hbm, h_v)

        def out_dst(hbm, l, jt):
            return hbm.at[pl.ds(((l * 2 + jt) * NBT + bt0) * 1024, half)]

        def drain(p, l):
            for jt in range(2):
                pltpu.make_async_copy(
                    hbufs[p].at[pl.ds(jt * half, half)],
                    out_dst(hid_hbm, l, jt), wsems[p]).wait()
                pltpu.make_async_copy(
                    lbufs[p].at[pl.ds(jt * half, half)],
                    out_dst(log_hbm, l, jt), wsems[p]).wait()

        def lt_body(lt, carry):
            pltpu.sync_copy(
                ids_hbm.at[pl.ds((lt * NBT + bt0) * 1024, blk)], ids_v)
            for ll in range(8):
                p = ll % 2
                l = lt * 8 + ll
                hid_l, log_l = hbufs[p], lbufs[p]

                if ll >= 2:
                    drain(p, l)
                else:
                    @pl.when(lt >= 1)
                    def _():
                        drain(p, l)

                @plsc.parallel_loop(0, btw * 8, unroll=4)
                def _(t):
                    base_t = (t // 8) * 1024 + (t % 8) * 16
                    idv = ids_v[pl.ds(base_t + ll * 128, 16)]
                    bi = idv * D
                    rows = [plsc.load_gather(h_v, (bi + j,) if j else (bi,))
                            for j in range(D)]
                    g0 = rows[0]
                    for j in range(D):
                        off = (j // 8) * half + base_t + (j % 8) * 128
                        hid_l[pl.ds(off, 16)] = rows[j]
                        log_l[pl.ds(off, 16)] = g0

                for jt in range(2):
                    pltpu.async_copy(
                        hid_l.at[pl.ds(jt * half, half)],
                        out_dst(hid_hbm, l, jt), wsems[p])
                    pltpu.async_copy(
                        log_l.at[pl.ds(jt * half, half)],
                        out_dst(log_hbm, l, jt), wsems[p])
            return carry

        lax.fori_loop(0, LT, lt_body, 0, unroll=False)

        for ll in (6, 7):
            drain(ll % 2, (LT - 1) * 8 + ll)

    return k(ids_phys, h_flat)


def kernel(input_ids, embed_table, W, b):
    ids_phys = (input_ids.T.reshape(LT, 8, NBT, 128)
                .transpose(0, 2, 1, 3).reshape(-1).astype(jnp.int32))
    h = _compute_h(embed_table, W, b)
    hid_flat, log_flat = _sc_gather(ids_phys, h.reshape(-1))

    def unphys(flat):
        return (flat.reshape(L, 2, NBT, 8, 128)
                .transpose(2, 4, 0, 1, 3).reshape(B, L, D))

    return (unphys(log_flat), unphys(hid_flat))


# speedup vs baseline: 11.0090x; 1.0240x over previous
import functools

import jax
import jax.numpy as jnp
from jax import lax
from jax.experimental import pallas as pl
from jax.experimental.pallas import tpu as pltpu
from jax.experimental.pallas import tpu_sc as plsc

VOCAB = 16
D = 16
B = 16384
L = 200
LT = L // 8
NBT = B // 128


def _h_body(e_ref, w_ref, b_ref, h_ref):
    h = lax.dot_general(
        e_ref[...], w_ref[...],
        (((1,), (1,)), ((), ())),
        preferred_element_type=jnp.float32,
    )
    h_ref[...] = h + b_ref[...]


def _compute_h(embed_table, W, b):
    b_mat = jnp.broadcast_to(b.reshape(1, D), (VOCAB, D))
    return pl.pallas_call(
        _h_body,
        out_shape=jax.ShapeDtypeStruct((VOCAB, D), jnp.float32),
    )(embed_table, W, b_mat)


def _sc_gather(ids_phys, h_flat):
    n = ids_phys.shape[0]
    assert n == B * L
    info = plsc.get_sparse_core_info()
    nc, ns = info.num_cores, info.num_subcores
    nw = nc * ns
    btw = NBT // nw
    assert btw * nw == NBT
    blk = btw * 8 * 128

    mesh = plsc.VectorSubcoreMesh(core_axis_name="c", subcore_axis_name="s")

    @functools.partial(
        pl.kernel,
        out_type=[
            jax.ShapeDtypeStruct((n * D,), jnp.float32),
            jax.ShapeDtypeStruct((n * D,), jnp.float32),
        ],
        mesh=mesh,
        scratch_types=[
            pltpu.VMEM((VOCAB * D,), jnp.float32),
            pltpu.VMEM((blk,), jnp.int32),
            pltpu.VMEM((2 * btw * 1024,), jnp.float32),
            pltpu.VMEM((2 * btw * 1024,), jnp.float32),
            pltpu.VMEM((2 * btw * 1024,), jnp.float32),
            pltpu.VMEM((2 * btw * 1024,), jnp.float32),
            pltpu.SemaphoreType.DMA,
            pltpu.SemaphoreType.DMA,
        ],
        compiler_params=pltpu.CompilerParams(
            needs_layout_passes=False, use_tc_tiling_on_sc=True),
    )
    def k(ids_hbm, h_hbm, hid_hbm, log_hbm,
          h_v, ids_v, hb0, hb1, lb0, lb1, sw0, sw1):
        wid = lax.axis_index("s") * nc + lax.axis_index("c")
        bt0 = wid * btw
        hbufs = (hb0, hb1)
        lbufs = (lb0, lb1)
        wsems = (sw0, sw1)
        half = btw * 1024

        pltpu.sync_copy(h_hbm, h_v)

        def out_dst(hbm, l, jt):
            return hbm.at[pl.ds(((l * 2 + jt) * NBT + bt0) * 1024, half)]

        def drain(p, l):
            for jt in range(2):
                pltpu.make_async_copy(
                    hbufs[p].at[pl.ds(jt * half, half)],
                    out_dst(hid_hbm, l, jt), wsems[p]).wait()
                pltpu.make_async_copy(
                    lbufs[p].at[pl.ds(jt * half, half)],
                    out_dst(log_hbm, l, jt), wsems[p]).wait()

        def lt_body(lt, carry):
            pltpu.sync_copy(
                ids_hbm.at[pl.ds((lt * NBT + bt0) * 1024, blk)], ids_v)
            for ll in range(8):
                p = ll % 2
                l = lt * 8 + ll
                hid_l, log_l = hbufs[p], lbufs[p]

                if ll >= 2:
                    drain(p, l)
                else:
                    @pl.when(lt >= 1)
                    def _():
                        drain(p, l)

                @plsc.parallel_loop(0, btw * 8, unroll=8)
                def _(t):
                    base_t = (t // 8) * 1024 + (t % 8) * 16
                    idv = ids_v[pl.ds(base_t + ll * 128, 16)]
                    bi = idv * D
                    rows = [plsc.load_gather(h_v, (bi + j,) if j else (bi,))
                            for j in range(D)]
                    g0 = rows[0]
                    for j in range(D):
                        off = (j // 8) * half + base_t + (j % 8) * 128
                        hid_l[pl.ds(off, 16)] = rows[j]
                        log_l[pl.ds(off, 16)] = g0

                for jt in range(2):
                    pltpu.async_copy(
                        hid_l.at[pl.ds(jt * half, half)],
                        out_dst(hid_hbm, l, jt), wsems[p])
                    pltpu.async_copy(
                        log_l.at[pl.ds(jt * half, half)],
                        out_dst(log_hbm, l, jt), wsems[p])
            return carry

        lax.fori_loop(0, LT, lt_body, 0, unroll=False)

        for ll in (6, 7):
            drain(ll % 2, (LT - 1) * 8 + ll)

    return k(ids_phys, h_flat)


def kernel(input_ids, embed_table, W, b):
    ids_phys = (input_ids.T.reshape(LT, 8, NBT, 128)
                .transpose(0, 2, 1, 3).reshape(-1).astype(jnp.int32))
    h = _compute_h(embed_table, W, b)
    hid_flat, log_flat = _sc_gather(ids_phys, h.reshape(-1))

    def unphys(flat):
        return (flat.reshape(L, 2, NBT, 8, 128)
                .transpose(2, 4, 0, 1, 3).reshape(B, L, D))

    return (unphys(log_flat), unphys(hid_flat))


# speedup vs baseline: 11.2197x; 1.0191x over previous
import functools

import jax
import jax.numpy as jnp
from jax import lax
from jax.experimental import pallas as pl
from jax.experimental.pallas import tpu as pltpu
from jax.experimental.pallas import tpu_sc as plsc

VOCAB = 16
D = 16
B = 16384
L = 200
LT = L // 8
NBT = B // 128


def _h_body(e_ref, w_ref, b_ref, h_ref):
    h = lax.dot_general(
        e_ref[...], w_ref[...],
        (((1,), (1,)), ((), ())),
        preferred_element_type=jnp.float32,
    )
    h_ref[...] = h + b_ref[...]


def _compute_h(embed_table, W, b):
    b_mat = jnp.broadcast_to(b.reshape(1, D), (VOCAB, D))
    return pl.pallas_call(
        _h_body,
        out_shape=jax.ShapeDtypeStruct((VOCAB, D), jnp.float32),
    )(embed_table, W, b_mat)


def _sc_gather(ids_phys, h_flat):
    n = ids_phys.shape[0]
    assert n == B * L
    info = plsc.get_sparse_core_info()
    nc, ns = info.num_cores, info.num_subcores
    nw = nc * ns
    btw = NBT // nw
    assert btw * nw == NBT
    blk = btw * 8 * 128

    mesh = plsc.VectorSubcoreMesh(core_axis_name="c", subcore_axis_name="s")

    @functools.partial(
        pl.kernel,
        out_type=[
            jax.ShapeDtypeStruct((n * D,), jnp.float32),
            jax.ShapeDtypeStruct((n * D,), jnp.float32),
        ],
        mesh=mesh,
        scratch_types=[
            pltpu.VMEM((VOCAB * D,), jnp.float32),
            pltpu.VMEM((blk,), jnp.int32),
            pltpu.VMEM((blk,), jnp.int32),
            pltpu.VMEM((2 * btw * 1024,), jnp.float32),
            pltpu.VMEM((2 * btw * 1024,), jnp.float32),
            pltpu.VMEM((2 * btw * 1024,), jnp.float32),
            pltpu.VMEM((2 * btw * 1024,), jnp.float32),
            pltpu.SemaphoreType.DMA,
            pltpu.SemaphoreType.DMA,
            pltpu.SemaphoreType.DMA,
            pltpu.SemaphoreType.DMA,
        ],
        compiler_params=pltpu.CompilerParams(
            needs_layout_passes=False, use_tc_tiling_on_sc=True),
    )
    def k(ids_hbm, h_hbm, hid_hbm, log_hbm,
          h_v, iv0, iv1, hb0, hb1, lb0, lb1, si0, si1, sw0, sw1):
        wid = lax.axis_index("s") * nc + lax.axis_index("c")
        bt0 = wid * btw
        ivbufs = (iv0, iv1)
        isems = (si0, si1)
        hbufs = (hb0, hb1)
        lbufs = (lb0, lb1)
        wsems = (sw0, sw1)
        half = btw * 1024

        pltpu.sync_copy(h_hbm, h_v)

        def ids_src(lt):
            return ids_hbm.at[pl.ds((lt * NBT + bt0) * 1024, blk)]

        def out_dst(hbm, l, jt):
            return hbm.at[pl.ds(((l * 2 + jt) * NBT + bt0) * 1024, half)]

        def drain(p, l):
            for jt in range(2):
                pltpu.make_async_copy(
                    hbufs[p].at[pl.ds(jt * half, half)],
                    out_dst(hid_hbm, l, jt), wsems[p]).wait()
                pltpu.make_async_copy(
                    lbufs[p].at[pl.ds(jt * half, half)],
                    out_dst(log_hbm, l, jt), wsems[p]).wait()

        pltpu.async_copy(ids_src(0), iv0, si0)

        def do_lt(lt, d, prefetch, guard):
            pltpu.make_async_copy(ids_src(lt), ivbufs[d], isems[d]).wait()
            if prefetch:
                pltpu.async_copy(ids_src(lt + 1), ivbufs[1 - d],
                                 isems[1 - d])
            ids_v = ivbufs[d]
            for ll in range(8):
                p = ll % 2
                l = lt * 8 + ll
                hid_l, log_l = hbufs[p], lbufs[p]

                if ll >= 2 or guard is None:
                    drain(p, l)
                else:
                    @pl.when(guard)
                    def _():
                        drain(p, l)

                @plsc.parallel_loop(0, btw * 8, unroll=4)
                def _(t):
                    base_t = (t // 8) * 1024 + (t % 8) * 16
                    idv = ids_v[pl.ds(base_t + ll * 128, 16)]
                    bi = idv * D
                    rows = [plsc.load_gather(h_v, (bi + j,) if j else (bi,))
                            for j in range(D)]
                    g0 = rows[0]
                    for j in range(D):
                        off = (j // 8) * half + base_t + (j % 8) * 128
                        hid_l[pl.ds(off, 16)] = rows[j]
                        log_l[pl.ds(off, 16)] = g0

                for jt in range(2):
                    pltpu.async_copy(
                        hid_l.at[pl.ds(jt * half, half)],
                        out_dst(hid_hbm, l, jt), wsems[p])
                    pltpu.async_copy(
                        log_l.at[pl.ds(jt * half, half)],
                        out_dst(log_hbm, l, jt), wsems[p])

        def lt_body(i, carry):
            do_lt(i * 2, 0, True, i >= 1)
            do_lt(i * 2 + 1, 1, True, None)
            return carry

        lax.fori_loop(0, LT // 2, lt_body, 0, unroll=False)
        do_lt(LT - 1, 0, False, None)

        for ll in (6, 7):
            drain(ll % 2, (LT - 1) * 8 + ll)

    return k(ids_phys, h_flat)


def kernel(input_ids, embed_table, W, b):
    ids_phys = (input_ids.T.reshape(LT, 8, NBT, 128)
                .transpose(0, 2, 1, 3).reshape(-1).astype(jnp.int32))
    h = _compute_h(embed_table, W, b)
    hid_flat, log_flat = _sc_gather(ids_phys, h.reshape(-1))

    def unphys(flat):
        return (flat.reshape(L, 2, NBT, 8, 128)
                .transpose(2, 4, 0, 1, 3).reshape(B, L, D))

    return (unphys(log_flat), unphys(hid_flat))
